# baseline (speedup 1.0000x reference)
import functools

import jax
import jax.numpy as jnp
from jax import lax
from jax.experimental import pallas as pl
from jax.experimental.pallas import tpu as pltpu
from jax.experimental.pallas import tpu_sc as plsc

B, C, N, L = 4, 3, 512, 4096
BC = B * C
NW = 32
LANES = 16
WCOLS = 512
STRIPS = L // WCOLS
UNITS_PW = BC * STRIPS // NW
GRP = WCOLS // LANES
PASSES = 8
GPP = GRP // PASSES
RBLK = 64
NBLK = N // RBLK
NTASK = UNITS_PW * NBLK
NEG = float("-inf")


def _sc_topk(mixed, ref3):
    mesh = plsc.VectorSubcoreMesh(core_axis_name="c", subcore_axis_name="s")

    @functools.partial(
        pl.kernel,
        mesh=mesh,
        out_type=[
            jax.ShapeDtypeStruct((BC, 2, L), jnp.float32),
            jax.ShapeDtypeStruct((BC, L), jnp.int32),
        ],
        scratch_types=[
            pltpu.VMEM((B, L), jnp.float32),
            pltpu.VMEM((2, RBLK, WCOLS), jnp.float32),
            pltpu.VMEM((WCOLS,), jnp.float32),
            pltpu.VMEM((WCOLS,), jnp.float32),
            pltpu.VMEM((WCOLS,), jnp.int32),
            pltpu.SemaphoreType.DMA,
            pltpu.SemaphoreType.DMA,
        ],
    )
    def k(mixed_hbm, ref_hbm, maxs_hbm, idxs_hbm, m_v, buf_v, sm1, sm2, six,
          sem0, sem1):
        wid = lax.axis_index("s") * 2 + lax.axis_index("c")
        sems = (sem0, sem1)

        def task_copy(t, slot):
            unit = wid * UNITS_PW + t // NBLK
            blk = t % NBLK
            bc = unit // STRIPS
            col0 = (unit % STRIPS) * WCOLS
            return pltpu.make_async_copy(
                ref_hbm.at[bc, pl.ds(blk * RBLK, RBLK), pl.ds(col0, WCOLS)],
                buf_v.at[slot],
                sems[slot],
            )

        pltpu.sync_copy(mixed_hbm, m_v)
        task_copy(0, 0).start()
        task_copy(1, 1).start()

        def outer(tp, carry):
            for u in range(2):
                t = tp * 2 + u
                unit = wid * UNITS_PW + t // NBLK
                blk = t % NBLK
                bc = unit // STRIPS
                col0 = (unit % STRIPS) * WCOLS
                b = bc // C
                task_copy(t, u).wait()

                @pl.when(blk == 0)
                def _init():
                    neg = jnp.full((LANES,), NEG, jnp.float32)
                    zero = jnp.zeros((LANES,), jnp.int32)
                    for j in range(GRP):
                        sm1[pl.ds(j * LANES, LANES)] = neg
                        sm2[pl.ds(j * LANES, LANES)] = neg
                        six[pl.ds(j * LANES, LANES)] = zero

                for p in range(PASSES):
                    g0 = p * GPP
                    st = []
                    for g in range(GPP):
                        st.append(sm1[pl.ds((g0 + g) * LANES, LANES)])
                        st.append(sm2[pl.ds((g0 + g) * LANES, LANES)])
                        st.append(six[pl.ds((g0 + g) * LANES, LANES)])
                    mv = [m_v[b, pl.ds(col0 + (g0 + g) * LANES, LANES)]
                          for g in range(GPP)]

                    def inner(i, s, g0=g0, mv=mv, blk=blk, u=u):
                        r0 = i * 2
                        nv0 = jnp.full((LANES,), blk * RBLK + r0, jnp.int32)
                        nv1 = nv0 + 1
                        out = []
                        for g in range(GPP):
                            m1 = s[3 * g]
                            m2 = s[3 * g + 1]
                            ix = s[3 * g + 2]
                            cs = pl.ds((g0 + g) * LANES, LANES)
                            va = buf_v[u, r0, cs] * mv[g]
                            vb = buf_v[u, r0 + 1, cs] * mv[g]
                            hi = jnp.maximum(va, vb)
                            lo = jnp.minimum(va, vb)
                            ihm = jnp.where(va >= vb, nv0, nv1)
                            gt = hi > m1
                            ix = jnp.where(gt, ihm, ix)
                            t1 = jnp.minimum(m1, hi)
                            m1 = jnp.maximum(m1, hi)
                            m2 = jnp.maximum(t1, jnp.maximum(m2, lo))
                            out += [m1, m2, ix]
                        return tuple(out)

                    fin = lax.fori_loop(0, RBLK // 2, inner, tuple(st))
                    for g in range(GPP):
                        sm1[pl.ds((g0 + g) * LANES, LANES)] = fin[3 * g]
                        sm2[pl.ds((g0 + g) * LANES, LANES)] = fin[3 * g + 1]
                        six[pl.ds((g0 + g) * LANES, LANES)] = fin[3 * g + 2]

                nt = t + 2

                @pl.when(nt < NTASK)
                def _start_next(t=nt, u=u):
                    task_copy(t, u).start()

                @pl.when(blk == NBLK - 1)
                def _flush(bc=bc, col0=col0):
                    pltpu.sync_copy(sm1, maxs_hbm.at[bc, 0, pl.ds(col0, WCOLS)])
                    pltpu.sync_copy(sm2, maxs_hbm.at[bc, 1, pl.ds(col0, WCOLS)])
                    pltpu.sync_copy(six, idxs_hbm.at[bc, pl.ds(col0, WCOLS)])

            return carry

        lax.fori_loop(0, NTASK // 2, outer, 0)

    return k(mixed, ref3)


def kernel(input_mixed, ref_panel):
    ref3 = ref_panel.reshape(BC, N, L)
    maxs, idxs = _sc_topk(input_mixed, ref3)
    return maxs.reshape(B, C, 2, L), idxs.reshape(B, C, L)


# speedup vs baseline: 1.3632x; 1.3632x over previous
import functools

import jax
import jax.numpy as jnp
from jax import lax
from jax.experimental import pallas as pl
from jax.experimental.pallas import tpu as pltpu
from jax.experimental.pallas import tpu_sc as plsc

B, C, N, L = 4, 3, 512, 4096
BC = B * C
P_TC = 8
P_SC = BC - P_TC
NW = 32
LANES = 16
WCOLS = 512
STRIPS = L // WCOLS
GRP = WCOLS // LANES
PASSES = 4
GPP = GRP // PASSES
RBLK = 64
NBLK = N // RBLK
NEG = float("-inf")
LT = 1024


def _sc_topk(mixed, ref3):
    mesh = plsc.VectorSubcoreMesh(core_axis_name="c", subcore_axis_name="s")

    @functools.partial(
        pl.kernel,
        mesh=mesh,
        out_type=[
            jax.ShapeDtypeStruct((P_SC, 2, L), jnp.float32),
            jax.ShapeDtypeStruct((P_SC, L), jnp.int32),
        ],
        scratch_types=[
            pltpu.VMEM((B, L), jnp.float32),
            pltpu.VMEM((2, RBLK, WCOLS), jnp.float32),
            pltpu.VMEM((WCOLS,), jnp.float32),
            pltpu.VMEM((WCOLS,), jnp.float32),
            pltpu.VMEM((WCOLS,), jnp.int32),
            pltpu.SemaphoreType.DMA,
            pltpu.SemaphoreType.DMA,
        ],
    )
    def k(mixed_hbm, ref_hbm, maxs_hbm, idxs_hbm, m_v, buf_v, sm1, sm2, six,
          sem0, sem1):
        wid = lax.axis_index("s") * 2 + lax.axis_index("c")
        pair = wid // STRIPS
        bc = P_TC + pair
        col0 = (wid % STRIPS) * WCOLS
        b = bc // C
        sems = (sem0, sem1)

        def blk_copy(blk, slot):
            return pltpu.make_async_copy(
                ref_hbm.at[bc, pl.ds(blk * RBLK, RBLK), pl.ds(col0, WCOLS)],
                buf_v.at[slot],
                sems[slot],
            )

        pltpu.sync_copy(mixed_hbm, m_v)
        blk_copy(0, 0).start()
        blk_copy(1, 1).start()

        neg = jnp.full((LANES,), NEG, jnp.float32)
        zero = jnp.zeros((LANES,), jnp.int32)
        for j in range(GRP):
            sm1[pl.ds(j * LANES, LANES)] = neg
            sm2[pl.ds(j * LANES, LANES)] = neg
            six[pl.ds(j * LANES, LANES)] = zero

        def outer(bp, carry):
            for u in range(2):
                blk = bp * 2 + u
                blk_copy(blk, u).wait()

                for p in range(PASSES):
                    g0 = p * GPP
                    st = []
                    for g in range(GPP):
                        st.append(sm1[pl.ds((g0 + g) * LANES, LANES)])
                        st.append(sm2[pl.ds((g0 + g) * LANES, LANES)])
                        st.append(six[pl.ds((g0 + g) * LANES, LANES)])
                    mv = [m_v[b, pl.ds(col0 + (g0 + g) * LANES, LANES)]
                          for g in range(GPP)]

                    def inner(i, s, g0=g0, mv=mv, blk=blk, u=u):
                        r0 = i * 2
                        nv0 = jnp.full((LANES,), blk * RBLK + r0, jnp.int32)
                        nv1 = nv0 + 1
                        out = []
                        for g in range(GPP):
                            m1 = s[3 * g]
                            m2 = s[3 * g + 1]
                            ix = s[3 * g + 2]
                            cs = pl.ds((g0 + g) * LANES, LANES)
                            va = buf_v[u, r0, cs] * mv[g]
                            vb = buf_v[u, r0 + 1, cs] * mv[g]
                            hi = jnp.maximum(va, vb)
                            lo = jnp.minimum(va, vb)
                            ihm = jnp.where(va >= vb, nv0, nv1)
                            gt = hi > m1
                            ix = jnp.where(gt, ihm, ix)
                            t1 = jnp.minimum(m1, hi)
                            m1 = jnp.maximum(m1, hi)
                            m2 = jnp.maximum(t1, jnp.maximum(m2, lo))
                            out += [m1, m2, ix]
                        return tuple(out)

                    fin = lax.fori_loop(0, RBLK // 2, inner, tuple(st))
                    for g in range(GPP):
                        sm1[pl.ds((g0 + g) * LANES, LANES)] = fin[3 * g]
                        sm2[pl.ds((g0 + g) * LANES, LANES)] = fin[3 * g + 1]
                        six[pl.ds((g0 + g) * LANES, LANES)] = fin[3 * g + 2]

                nblk = blk + 2

                @pl.when(nblk < NBLK)
                def _start_next(nblk=nblk, u=u):
                    blk_copy(nblk, u).start()

            return carry

        lax.fori_loop(0, NBLK // 2, outer, 0)

        pltpu.sync_copy(sm1, maxs_hbm.at[pair, 0, pl.ds(col0, WCOLS)])
        pltpu.sync_copy(sm2, maxs_hbm.at[pair, 1, pl.ds(col0, WCOLS)])
        pltpu.sync_copy(six, idxs_hbm.at[pair, pl.ds(col0, WCOLS)])

    return k(mixed, ref3)


def _tc_body(m_ref, r_ref, v_ref, i_ref):
    m = m_ref[0, 0, :]
    prod = r_ref[0] * m[None, :]
    max1 = jnp.max(prod, axis=0)
    iota = lax.broadcasted_iota(jnp.int32, (N, LT), 0)
    cand = jnp.where(prod == max1[None, :], iota, N)
    amax = jnp.min(cand, axis=0)
    masked = jnp.where(iota == amax[None, :], NEG, prod)
    max2 = jnp.max(masked, axis=0)
    v_ref[0, 0, :] = max1
    v_ref[0, 1, :] = max2
    i_ref[0, 0, :] = amax


def _tc_topk(mixed, ref3):
    return pl.pallas_call(
        _tc_body,
        grid=(P_TC, L // LT),
        in_specs=[
            pl.BlockSpec((1, 1, LT), lambda i, j: (i // C, 0, j)),
            pl.BlockSpec((1, N, LT), lambda i, j: (i, 0, j)),
        ],
        out_specs=[
            pl.BlockSpec((1, 2, LT), lambda i, j: (i, 0, j)),
            pl.BlockSpec((1, 1, LT), lambda i, j: (i, 0, j)),
        ],
        out_shape=[
            jax.ShapeDtypeStruct((P_TC, 2, L), jnp.float32),
            jax.ShapeDtypeStruct((P_TC, 1, L), jnp.int32),
        ],
    )(mixed.reshape(B, 1, L), ref3)


def kernel(input_mixed, ref_panel):
    ref3 = ref_panel.reshape(BC, N, L)
    tcv, tci = _tc_topk(input_mixed, ref3)
    scv, sci = _sc_topk(input_mixed, ref3)
    maxs = jnp.concatenate([tcv, scv], axis=0).reshape(B, C, 2, L)
    idxs = jnp.concatenate([tci[:, 0, :], sci], axis=0).reshape(B, C, L)
    return maxs, idxs


# speedup vs baseline: 1.3647x; 1.0011x over previous
import functools

import jax
import jax.numpy as jnp
from jax import lax
from jax.experimental import pallas as pl
from jax.experimental.pallas import tpu as pltpu
from jax.experimental.pallas import tpu_sc as plsc

B, C, N, L = 4, 3, 512, 4096
BC = B * C
P_TC = 8
P_SC = BC - P_TC
NW = 32
LANES = 16
WCOLS = 512
STRIPS = L // WCOLS
GRP = WCOLS // LANES
PASSES = 4
GPP = GRP // PASSES
RBLK = 64
NBLK = N // RBLK
NEG = float("-inf")
LT = 1024


def _sc_topk(mixed, ref3):
    mesh = plsc.VectorSubcoreMesh(core_axis_name="c", subcore_axis_name="s")

    @functools.partial(
        pl.kernel,
        mesh=mesh,
        out_type=[
            jax.ShapeDtypeStruct((P_SC, 2, L), jnp.float32),
            jax.ShapeDtypeStruct((P_SC, L), jnp.int32),
        ],
        scratch_types=[
            pltpu.VMEM((B, L), jnp.float32),
            pltpu.VMEM((2, RBLK, WCOLS), jnp.float32),
            pltpu.VMEM((WCOLS,), jnp.float32),
            pltpu.VMEM((WCOLS,), jnp.float32),
            pltpu.VMEM((WCOLS,), jnp.int32),
            pltpu.SemaphoreType.DMA,
            pltpu.SemaphoreType.DMA,
        ],
    )
    def k(mixed_hbm, ref_hbm, maxs_hbm, idxs_hbm, m_v, buf_v, sm1, sm2, six,
          sem0, sem1):
        wid = lax.axis_index("s") * 2 + lax.axis_index("c")
        pair = wid // STRIPS
        bc = P_TC + pair
        col0 = (wid % STRIPS) * WCOLS
        b = bc // C
        sems = (sem0, sem1)

        def blk_copy(blk, slot):
            return pltpu.make_async_copy(
                ref_hbm.at[bc, pl.ds(blk * RBLK, RBLK), pl.ds(col0, WCOLS)],
                buf_v.at[slot],
                sems[slot],
            )

        pltpu.sync_copy(mixed_hbm, m_v)
        blk_copy(0, 0).start()
        blk_copy(1, 1).start()

        neg = jnp.full((LANES,), NEG, jnp.float32)
        zero = jnp.zeros((LANES,), jnp.int32)
        for j in range(GRP):
            sm1[pl.ds(j * LANES, LANES)] = neg
            sm2[pl.ds(j * LANES, LANES)] = neg
            six[pl.ds(j * LANES, LANES)] = zero

        def outer(bp, carry):
            for u in range(2):
                blk = bp * 2 + u
                blk_copy(blk, u).wait()

                for p in range(PASSES):
                    g0 = p * GPP
                    st = []
                    for g in range(GPP):
                        st.append(sm1[pl.ds((g0 + g) * LANES, LANES)])
                        st.append(sm2[pl.ds((g0 + g) * LANES, LANES)])
                        st.append(six[pl.ds((g0 + g) * LANES, LANES)])
                    mv = [m_v[b, pl.ds(col0 + (g0 + g) * LANES, LANES)]
                          for g in range(GPP)]

                    def inner(i, s, g0=g0, mv=mv, blk=blk, u=u):
                        r0 = i * 2
                        nv0 = jnp.full((LANES,), blk * RBLK + r0, jnp.int32)
                        nv1 = nv0 + 1
                        out = []
                        for g in range(GPP):
                            m1 = s[3 * g]
                            m2 = s[3 * g + 1]
                            ix = s[3 * g + 2]
                            cs = pl.ds((g0 + g) * LANES, LANES)
                            va = buf_v[u, r0, cs] * mv[g]
                            vb = buf_v[u, r0 + 1, cs] * mv[g]
                            hi = jnp.maximum(va, vb)
                            lo = jnp.minimum(va, vb)
                            ihm = jnp.where(va >= vb, nv0, nv1)
                            gt = hi > m1
                            ix = jnp.where(gt, ihm, ix)
                            t1 = jnp.minimum(m1, hi)
                            m1 = jnp.maximum(m1, hi)
                            m2 = jnp.maximum(t1, jnp.maximum(m2, lo))
                            out += [m1, m2, ix]
                        return tuple(out)

                    fin = lax.fori_loop(0, RBLK // 2, inner, tuple(st))
                    for g in range(GPP):
                        sm1[pl.ds((g0 + g) * LANES, LANES)] = fin[3 * g]
                        sm2[pl.ds((g0 + g) * LANES, LANES)] = fin[3 * g + 1]
                        six[pl.ds((g0 + g) * LANES, LANES)] = fin[3 * g + 2]

                nblk = blk + 2

                @pl.when(nblk < NBLK)
                def _start_next(nblk=nblk, u=u):
                    blk_copy(nblk, u).start()

            return carry

        lax.fori_loop(0, NBLK // 2, outer, 0)

        pltpu.sync_copy(sm1, maxs_hbm.at[pair, 0, pl.ds(col0, WCOLS)])
        pltpu.sync_copy(sm2, maxs_hbm.at[pair, 1, pl.ds(col0, WCOLS)])
        pltpu.sync_copy(six, idxs_hbm.at[pair, pl.ds(col0, WCOLS)])

    return k(mixed, ref3)


def _tc_body(m_ref, r_ref, v_ref, i_ref):
    m = m_ref[0, 0, :]
    prod = r_ref[0] * m[None, :]
    max1 = jnp.max(prod, axis=0)
    iota = lax.broadcasted_iota(jnp.int32, (N, LT), 0)
    cand = jnp.where(prod == max1[None, :], iota, N)
    amax = jnp.min(cand, axis=0)
    masked = jnp.where(iota == amax[None, :], NEG, prod)
    max2 = jnp.max(masked, axis=0)
    v_ref[0, 0, :] = max1
    v_ref[0, 1, :] = max2
    i_ref[0, 0, :] = amax


def _tc_topk(mixed, ref3):
    return pl.pallas_call(
        _tc_body,
        grid=(P_TC, L // LT),
        in_specs=[
            pl.BlockSpec((1, 1, LT), lambda i, j: (i // C, 0, j)),
            pl.BlockSpec((1, N, LT), lambda i, j: (i, 0, j)),
        ],
        out_specs=[
            pl.BlockSpec((1, 2, LT), lambda i, j: (i, 0, j)),
            pl.BlockSpec((1, 1, LT), lambda i, j: (i, 0, j)),
        ],
        out_shape=[
            jax.ShapeDtypeStruct((P_TC, 2, L), jnp.float32),
            jax.ShapeDtypeStruct((P_TC, 1, L), jnp.int32),
        ],
    )(mixed.reshape(B, 1, L), ref3)


def kernel(input_mixed, ref_panel):
    ref3 = ref_panel.reshape(BC, N, L)
    scv, sci = _sc_topk(input_mixed, ref3)
    tcv, tci = _tc_topk(input_mixed, ref3)
    maxs = jnp.concatenate([tcv, scv], axis=0).reshape(B, C, 2, L)
    idxs = jnp.concatenate([tci[:, 0, :], sci], axis=0).reshape(B, C, L)
    return maxs, idxs
